# Initial kernel scaffold; baseline (speedup 1.0000x reference)
import jax
import jax.numpy as jnp
from jax.experimental import pallas as pl


def kernel(batch_size, max_length, encoder_hidden, emb, W_ih, W_hh, b_ih, b_hh, W_out, b_out):
    raise NotImplementedError("write your pallas kernel here")


# speedup vs baseline: 3.1037x; 3.1037x over previous
import functools

import jax
import jax.numpy as jnp
from jax.experimental import pallas as pl
from jax.experimental.pallas import tpu as pltpu

V = 100000
H = 128
K = 8
SOS = 1
EOS = 2
NEG = -1e9

SV = 2048
V_PAD = 102400
NV = V_PAD // SV
BK = 128
NEGBIG = -3e38
INTBIG = 2**31 - 1


def _extract_top8(vals, idxs):
    tv, ti = [], []
    for _ in range(K):
        mx = jnp.max(vals, axis=1, keepdims=True)
        im = jnp.min(jnp.where(vals >= mx, idxs, INTBIG), axis=1, keepdims=True)
        tv.append(mx)
        ti.append(im)
        vals = jnp.where(idxs == im, NEGBIG, vals)
    return jnp.concatenate(tv, axis=1), jnp.concatenate(ti, axis=1)


def _step_kernel(x_ref, h_ref, wih_ref, whh_ref, bih_ref, bhh_ref,
                 wout_ref, bout_ref,
                 logits_ref, hnew_ref, topv_ref, topi_ref, lse_ref,
                 m_ref, s_ref):
    v = pl.program_id(0)
    nv = pl.num_programs(0)

    @pl.when(v == 0)
    def _init():
        x = x_ref[...]
        h = h_ref[...]
        gi = jnp.dot(x, wih_ref[...], preferred_element_type=jnp.float32) + bih_ref[...]
        gh = jnp.dot(h, whh_ref[...], preferred_element_type=jnp.float32) + bhh_ref[...]
        i_r, i_z, i_n = gi[:, :H], gi[:, H:2 * H], gi[:, 2 * H:]
        h_r, h_z, h_n = gh[:, :H], gh[:, H:2 * H], gh[:, 2 * H:]
        r = jax.nn.sigmoid(i_r + h_r)
        z = jax.nn.sigmoid(i_z + h_z)
        n = jnp.tanh(i_n + r * h_n)
        hnew_ref[...] = (1.0 - z) * n + z * h
        m_ref[...] = jnp.full((BK, 1), NEGBIG, dtype=jnp.float32)
        s_ref[...] = jnp.zeros((BK, 1), dtype=jnp.float32)
        topv_ref[...] = jnp.full((BK, K), NEGBIG, dtype=jnp.float32)
        topi_ref[...] = jnp.zeros((BK, K), dtype=jnp.int32)

    h = hnew_ref[...]
    logits = jnp.dot(h, wout_ref[...], preferred_element_type=jnp.float32) + bout_ref[...]
    logits_ref[...] = logits

    sm = jnp.max(logits, axis=1, keepdims=True)
    m_old = m_ref[...]
    m_new = jnp.maximum(m_old, sm)
    s_ref[...] = (s_ref[...] * jnp.exp(m_old - m_new)
                  + jnp.sum(jnp.exp(logits - m_new), axis=1, keepdims=True))
    m_ref[...] = m_new

    base = v * SV
    colidx = jax.lax.broadcasted_iota(jnp.int32, (BK, SV), 1) + base
    sv_v, sv_i = _extract_top8(logits, colidx)
    mv = jnp.concatenate([sv_v, topv_ref[...]], axis=1)
    mi = jnp.concatenate([sv_i, topi_ref[...]], axis=1)
    nt_v, nt_i = _extract_top8(mv, mi)
    topv_ref[...] = nt_v
    topi_ref[...] = nt_i

    @pl.when(v == nv - 1)
    def _fin():
        lse = m_ref[...] + jnp.log(s_ref[...])
        lse_ref[...] = jnp.broadcast_to(lse, (BK, K))


def _run_step(x, h, W_ih, W_hh, b_ih, b_hh, W_out_p, b_out_p):
    f32 = jnp.float32
    out_shapes = (
        jax.ShapeDtypeStruct((BK, V_PAD), f32),
        jax.ShapeDtypeStruct((BK, H), f32),
        jax.ShapeDtypeStruct((BK, K), f32),
        jax.ShapeDtypeStruct((BK, K), jnp.int32),
        jax.ShapeDtypeStruct((BK, K), f32),
    )
    const = lambda v: (0, 0)
    return pl.pallas_call(
        _step_kernel,
        grid=(NV,),
        in_specs=[
            pl.BlockSpec((BK, H), const),
            pl.BlockSpec((BK, H), const),
            pl.BlockSpec((H, 3 * H), const),
            pl.BlockSpec((H, 3 * H), const),
            pl.BlockSpec((1, 3 * H), const),
            pl.BlockSpec((1, 3 * H), const),
            pl.BlockSpec((H, SV), lambda v: (0, v)),
            pl.BlockSpec((1, SV), lambda v: (0, v)),
        ],
        out_specs=(
            pl.BlockSpec((BK, SV), lambda v: (0, v)),
            pl.BlockSpec((BK, H), const),
            pl.BlockSpec((BK, K), const),
            pl.BlockSpec((BK, K), const),
            pl.BlockSpec((BK, K), const),
        ),
        out_shape=out_shapes,
        scratch_shapes=[
            pltpu.VMEM((BK, 1), f32),
            pltpu.VMEM((BK, 1), f32),
        ],
        compiler_params=pltpu.CompilerParams(
            dimension_semantics=("arbitrary",)),
    )(x, h, W_ih, W_hh, b_ih, b_hh, W_out_p, b_out_p)


def _gather_kernel(sel_ref, logits_ref, lse_ref, out_ref):
    del sel_ref
    out_ref[...] = logits_ref[...] - lse_ref[0, 0, 0]


def _gather_step(logits, lse_sel, sel):
    B = sel.shape[0]
    logits4 = logits.reshape(BK, NV, 1, SV)
    grid_spec = pltpu.PrefetchScalarGridSpec(
        num_scalar_prefetch=1,
        grid=(B, NV),
        in_specs=[
            pl.BlockSpec((1, 1, 1, SV), lambda b, v, sel_ref: (sel_ref[b], v, 0, 0)),
            pl.BlockSpec((1, 1, H), lambda b, v, sel_ref: (b, 0, 0)),
        ],
        out_specs=pl.BlockSpec((1, 1, 1, SV), lambda b, v, sel_ref: (b, v, 0, 0)),
    )
    lse3 = jnp.broadcast_to(lse_sel[:, None, None], (B, 1, H))
    out4 = pl.pallas_call(
        _gather_kernel,
        grid_spec=grid_spec,
        out_shape=jax.ShapeDtypeStruct((B, NV, 1, SV), jnp.float32),
        compiler_params=pltpu.CompilerParams(
            dimension_semantics=("arbitrary", "arbitrary")),
    )(sel, logits4, lse3)
    return out4.reshape(B, V_PAD)


def kernel(batch_size, max_length, encoder_hidden, emb, W_ih, W_hh, b_ih, b_hh, W_out, b_out):
    B = encoder_hidden.shape[1]
    T = 8
    f32 = jnp.float32
    pos_index = jnp.arange(B) * K + (jnp.asarray(batch_size) - B).astype(jnp.int32)

    W_out_p = jnp.pad(W_out, ((0, 0), (0, V_PAD - V)))
    b_out_p = jnp.pad(b_out, (0, V_PAD - V), constant_values=NEGBIG / 2).reshape(1, V_PAD)
    b_ih2 = b_ih.reshape(1, 3 * H)
    b_hh2 = b_hh.reshape(1, 3 * H)

    h = jnp.repeat(encoder_hidden[0], K, axis=0)
    seq_scores = jnp.full((BK,), NEG, dtype=f32).at[pos_index].set(0.0)
    input_var = jnp.full((BK,), SOS, dtype=jnp.int32) + (jnp.asarray(max_length) - T).astype(jnp.int32)

    stored_logits = []
    stored_lse = []
    stored_pred = []
    for _ in range(T):
        x = jnp.take(emb, input_var, axis=0)
        logits, h_new, topv, topi, lse_b = _run_step(
            x, h, W_ih, W_hh, b_ih2, b_hh2, W_out_p, b_out_p)
        lse = lse_b[:, 0]
        stored_logits.append(logits)
        stored_lse.append(lse)

        total = seq_scores[:, None] + topv - lse[:, None]
        cand_val = total.reshape(B, K * K)
        scores, cidx = jax.lax.top_k(cand_val, K)
        src_beam = cidx // K
        src_slot = cidx % K
        rows = (pos_index[:, None] + src_beam).reshape(BK)
        input_var = topi[rows, src_slot.reshape(BK)].astype(jnp.int32)
        predecessors = rows
        stored_pred.append(predecessors)
        seq_scores = scores.reshape(BK)
        h = h_new[predecessors]
        seq_scores = jnp.where(input_var == EOS, NEG, seq_scores)

    sorted_score, sorted_idx = jax.lax.top_k(seq_scores.reshape(B, K), K)
    t_pred = (sorted_idx + pos_index[:, None]).reshape(BK)
    outs = []
    for t in range(T - 1, -1, -1):
        sel = t_pred.reshape(B, K)[:, 0]
        out_t = _gather_step(stored_logits[t], stored_lse[t][sel], sel)
        outs.append(out_t)
        t_pred = stored_pred[t][t_pred]
    outs = outs[::-1]
    decoder_outputs = jnp.stack(outs, axis=0)[:, :, :V]
    return decoder_outputs, sorted_score


# speedup vs baseline: 6.5397x; 2.1071x over previous
import functools

import jax
import jax.numpy as jnp
from jax.experimental import pallas as pl
from jax.experimental.pallas import tpu as pltpu

V = 100000
H = 128
K = 8
SOS = 1
EOS = 2
NEG = -1e9

SV = 2048
V_PAD = 102400
NV = V_PAD // SV
BK = 128
NEGBIG = -3e38
INTBIG = 2**31 - 1


def _extract_top8(vals, idxs):
    tv, ti = [], []
    for _ in range(K):
        mx = jnp.max(vals, axis=1, keepdims=True)
        im = jnp.min(jnp.where(vals >= mx, idxs, INTBIG), axis=1, keepdims=True)
        tv.append(mx)
        ti.append(im)
        vals = jnp.where(idxs == im, NEGBIG, vals)
    return jnp.concatenate(tv, axis=1), jnp.concatenate(ti, axis=1)


def _step_kernel(x_ref, h_ref, wih_ref, whh_ref, bih_ref, bhh_ref,
                 wout_ref, bout_ref,
                 logits_ref, hnew_ref, topv_ref, topi_ref, lse_ref,
                 m_ref, s_ref):
    v = pl.program_id(0)
    nv = pl.num_programs(0)

    @pl.when(v == 0)
    def _init():
        x = x_ref[...]
        h = h_ref[...]
        gi = jnp.dot(x, wih_ref[...], preferred_element_type=jnp.float32) + bih_ref[...]
        gh = jnp.dot(h, whh_ref[...], preferred_element_type=jnp.float32) + bhh_ref[...]
        i_r, i_z, i_n = gi[:, :H], gi[:, H:2 * H], gi[:, 2 * H:]
        h_r, h_z, h_n = gh[:, :H], gh[:, H:2 * H], gh[:, 2 * H:]
        r = jax.nn.sigmoid(i_r + h_r)
        z = jax.nn.sigmoid(i_z + h_z)
        n = jnp.tanh(i_n + r * h_n)
        hnew_ref[...] = (1.0 - z) * n + z * h
        m_ref[...] = jnp.full((BK, 1), NEGBIG, dtype=jnp.float32)
        s_ref[...] = jnp.zeros((BK, 1), dtype=jnp.float32)
        topv_ref[...] = jnp.full((BK, K), NEGBIG, dtype=jnp.float32)
        topi_ref[...] = jnp.zeros((BK, K), dtype=jnp.int32)

    h = hnew_ref[...]
    logits = jnp.dot(h, wout_ref[...], preferred_element_type=jnp.float32) + bout_ref[...]
    logits_ref[...] = logits

    sm = jnp.max(logits, axis=1, keepdims=True)
    m_old = m_ref[...]
    m_new = jnp.maximum(m_old, sm)
    s_ref[...] = (s_ref[...] * jnp.exp(m_old - m_new)
                  + jnp.sum(jnp.exp(logits - m_new), axis=1, keepdims=True))
    m_ref[...] = m_new

    base = v * SV
    colidx = jax.lax.broadcasted_iota(jnp.int32, (BK, SV), 1) + base
    sv_v, sv_i = _extract_top8(logits, colidx)
    mv = jnp.concatenate([sv_v, topv_ref[...]], axis=1)
    mi = jnp.concatenate([sv_i, topi_ref[...]], axis=1)
    nt_v, nt_i = _extract_top8(mv, mi)
    topv_ref[...] = nt_v
    topi_ref[...] = nt_i

    @pl.when(v == nv - 1)
    def _fin():
        lse = m_ref[...] + jnp.log(s_ref[...])
        lse_ref[...] = jnp.broadcast_to(lse, (BK, K))


def _run_step(x, h, W_ih, W_hh, b_ih, b_hh, W_out_p, b_out_p):
    f32 = jnp.float32
    out_shapes = (
        jax.ShapeDtypeStruct((BK, V_PAD), f32),
        jax.ShapeDtypeStruct((BK, H), f32),
        jax.ShapeDtypeStruct((BK, K), f32),
        jax.ShapeDtypeStruct((BK, K), jnp.int32),
        jax.ShapeDtypeStruct((BK, K), f32),
    )
    const = lambda v: (0, 0)
    return pl.pallas_call(
        _step_kernel,
        grid=(NV,),
        in_specs=[
            pl.BlockSpec((BK, H), const),
            pl.BlockSpec((BK, H), const),
            pl.BlockSpec((H, 3 * H), const),
            pl.BlockSpec((H, 3 * H), const),
            pl.BlockSpec((1, 3 * H), const),
            pl.BlockSpec((1, 3 * H), const),
            pl.BlockSpec((H, SV), lambda v: (0, v)),
            pl.BlockSpec((1, SV), lambda v: (0, v)),
        ],
        out_specs=(
            pl.BlockSpec((BK, SV), lambda v: (0, v)),
            pl.BlockSpec((BK, H), const),
            pl.BlockSpec((BK, K), const),
            pl.BlockSpec((BK, K), const),
            pl.BlockSpec((BK, K), const),
        ),
        out_shape=out_shapes,
        scratch_shapes=[
            pltpu.VMEM((BK, 1), f32),
            pltpu.VMEM((BK, 1), f32),
        ],
        compiler_params=pltpu.CompilerParams(
            dimension_semantics=("arbitrary",)),
    )(x, h, W_ih, W_hh, b_ih, b_hh, W_out_p, b_out_p)


def _gather_kernel(sel_ref, logits_ref, lse_ref, out_ref):
    del sel_ref
    out_ref[...] = logits_ref[...] - lse_ref[0, 0, 0]


def _gather_step(logits, lse_sel, sel):
    B = sel.shape[0]
    logits4 = logits.reshape(BK, 1, 1, V_PAD)
    grid_spec = pltpu.PrefetchScalarGridSpec(
        num_scalar_prefetch=1,
        grid=(B,),
        in_specs=[
            pl.BlockSpec((1, 1, 1, V_PAD), lambda b, sel_ref: (sel_ref[b], 0, 0, 0)),
            pl.BlockSpec((1, 1, H), lambda b, sel_ref: (b, 0, 0)),
        ],
        out_specs=pl.BlockSpec((1, 1, 1, V_PAD), lambda b, sel_ref: (b, 0, 0, 0)),
    )
    lse3 = jnp.broadcast_to(lse_sel[:, None, None], (B, 1, H))
    out4 = pl.pallas_call(
        _gather_kernel,
        grid_spec=grid_spec,
        out_shape=jax.ShapeDtypeStruct((B, 1, 1, V_PAD), jnp.float32),
        compiler_params=pltpu.CompilerParams(
            dimension_semantics=("arbitrary",)),
    )(sel, logits4, lse3)
    return out4.reshape(B, V_PAD)


def kernel(batch_size, max_length, encoder_hidden, emb, W_ih, W_hh, b_ih, b_hh, W_out, b_out):
    B = encoder_hidden.shape[1]
    T = 8
    f32 = jnp.float32
    pos_index = jnp.arange(B) * K + (jnp.asarray(batch_size) - B).astype(jnp.int32)

    W_out_p = jnp.pad(W_out, ((0, 0), (0, V_PAD - V)))
    b_out_p = jnp.pad(b_out, (0, V_PAD - V), constant_values=NEGBIG / 2).reshape(1, V_PAD)
    b_ih2 = b_ih.reshape(1, 3 * H)
    b_hh2 = b_hh.reshape(1, 3 * H)

    h = jnp.repeat(encoder_hidden[0], K, axis=0)
    seq_scores = jnp.full((BK,), NEG, dtype=f32).at[pos_index].set(0.0)
    input_var = jnp.full((BK,), SOS, dtype=jnp.int32) + (jnp.asarray(max_length) - T).astype(jnp.int32)

    stored_logits = []
    stored_lse = []
    stored_pred = []
    for _ in range(T):
        x = jnp.take(emb, input_var, axis=0)
        logits, h_new, topv, topi, lse_b = _run_step(
            x, h, W_ih, W_hh, b_ih2, b_hh2, W_out_p, b_out_p)
        lse = lse_b[:, 0]
        stored_logits.append(logits)
        stored_lse.append(lse)

        total = seq_scores[:, None] + topv - lse[:, None]
        cand_val = total.reshape(B, K * K)
        scores, cidx = jax.lax.top_k(cand_val, K)
        src_beam = cidx // K
        src_slot = cidx % K
        rows = (pos_index[:, None] + src_beam).reshape(BK)
        input_var = topi[rows, src_slot.reshape(BK)].astype(jnp.int32)
        predecessors = rows
        stored_pred.append(predecessors)
        seq_scores = scores.reshape(BK)
        h = h_new[predecessors]
        seq_scores = jnp.where(input_var == EOS, NEG, seq_scores)

    sorted_score, sorted_idx = jax.lax.top_k(seq_scores.reshape(B, K), K)
    t_pred = (sorted_idx + pos_index[:, None]).reshape(BK)
    outs = []
    for t in range(T - 1, -1, -1):
        sel = t_pred.reshape(B, K)[:, 0]
        out_t = _gather_step(stored_logits[t], stored_lse[t][sel], sel)
        outs.append(out_t)
        t_pred = stored_pred[t][t_pred]
    outs = outs[::-1]
    decoder_outputs = jnp.stack(outs, axis=0)[:, :, :V]
    return decoder_outputs, sorted_score
